# Initial kernel scaffold; baseline (speedup 1.0000x reference)
import jax
import jax.numpy as jnp
from jax import lax
from jax.experimental import pallas as pl
from jax.experimental.pallas import tpu as pltpu
from jax.experimental.pallas import tpu_sc as plsc

N_NODES = 50000
C = 64
HALF = 32
N_EDGES = 800000
EPS = 1e-5

NS = 16
CHUNK = 80
NBUF = 5
EPT = N_EDGES // NS
CPT = EPT // CHUNK
ROWS_PT = N_NODES // NS


def _sc_agg_body(h_a, h_b, src2, tgt2, f_a, f_b, g_a, g_b,
                 gidx, sidx, rows, acc, sem_g):
    c = lax.axis_index("c")
    s = lax.axis_index("s")

    def phase(h_ref, gat2, sct2, out_ref):
        pltpu.sync_copy(gat2.at[pl.ds(s * CPT, CPT)], gidx)
        pltpu.sync_copy(sct2.at[pl.ds(s * CPT, CPT)], sidx)
        pltpu.sync_copy(h_ref.at[pl.ds(s * ROWS_PT, ROWS_PT)],
                        acc.at[pl.ds(s * ROWS_PT, ROWS_PT)])
        plsc.subcore_barrier()

        for b in range(NBUF):
            pltpu.async_copy(h_ref.at[gidx.at[b]], rows.at[b], sem_g.at[b])

        def body(jo, carry):
            for b in range(NBUF):
                j = jo * NBUF + b
                pltpu.make_async_copy(
                    h_ref.at[gidx.at[j]], rows.at[b], sem_g.at[b]).wait()
                pltpu.sync_copy(rows.at[b], acc.at[sidx.at[j]], add=True)
                jn = j + NBUF

                @pl.when(jn < CPT)
                def _():
                    pltpu.async_copy(
                        h_ref.at[gidx.at[jn]], rows.at[b], sem_g.at[b])
            return carry

        lax.fori_loop(0, CPT // NBUF, body, 0)
        plsc.subcore_barrier()
        pltpu.sync_copy(acc.at[pl.ds(s * ROWS_PT, ROWS_PT)],
                        out_ref.at[pl.ds(s * ROWS_PT, ROWS_PT)])

    @pl.when(c == 0)
    def _():
        phase(h_a, src2, tgt2, f_a)
        phase(h_a, tgt2, src2, g_a)

    @pl.when(c == 1)
    def _():
        phase(h_b, src2, tgt2, f_b)
        phase(h_b, tgt2, src2, g_b)


_sc_agg = pl.kernel(
    _sc_agg_body,
    out_type=tuple(jax.ShapeDtypeStruct((N_NODES, HALF), jnp.float32)
                   for _ in range(4)),
    mesh=plsc.VectorSubcoreMesh(core_axis_name="c", subcore_axis_name="s"),
    scratch_types=[
        pltpu.VMEM((CPT, CHUNK), jnp.int32),
        pltpu.VMEM((CPT, CHUNK), jnp.int32),
        pltpu.VMEM((NBUF, CHUNK, HALF), jnp.float32),
        pltpu.VMEM_SHARED((N_NODES, HALF), jnp.float32),
        pltpu.SemaphoreType.DMA((NBUF,)),
    ],
)


BLK = 2000
NBLK = N_NODES // BLK


def _relu_stats(i, r, st_ref):
    s1 = jnp.sum(r, axis=0, keepdims=True)
    s2 = jnp.sum(r * r, axis=0, keepdims=True)
    blk = jnp.concatenate(
        [s1, s2, jnp.zeros((6, C), jnp.float32)], axis=0)

    @pl.when(i == 0)
    def _():
        st_ref[...] = blk

    @pl.when(i != 0)
    def _():
        st_ref[...] = st_ref[...] + blk


def _stats_body(x_ref, st_ref):
    _relu_stats(pl.program_id(0), jnp.maximum(x_ref[...], 0.0), st_ref)


_stats = pl.pallas_call(
    _stats_body,
    grid=(NBLK,),
    in_specs=[pl.BlockSpec((BLK, C), lambda i: (i, 0))],
    out_specs=pl.BlockSpec((8, C), lambda i: (0, 0)),
    out_shape=jax.ShapeDtypeStruct((8, C), jnp.float32),
)


def _bn_body(x_ref, st_ref, a_ref, b_ref):
    st = st_ref[...]
    mean = st[0:1, :] * (1.0 / N_NODES)
    var = st[1:2, :] * (1.0 / N_NODES) - mean * mean
    inv = lax.rsqrt(var + EPS)
    h = (jnp.maximum(x_ref[...], 0.0) - mean) * inv
    a_ref[...] = h[:, :HALF]
    b_ref[...] = h[:, HALF:]


_bn_apply = pl.pallas_call(
    _bn_body,
    grid=(NBLK,),
    in_specs=[
        pl.BlockSpec((BLK, C), lambda i: (i, 0)),
        pl.BlockSpec((8, C), lambda i: (0, 0)),
    ],
    out_specs=[
        pl.BlockSpec((BLK, HALF), lambda i: (i, 0)),
        pl.BlockSpec((BLK, HALF), lambda i: (i, 0)),
    ],
    out_shape=[jax.ShapeDtypeStruct((N_NODES, HALF), jnp.float32)
               for _ in range(2)],
)


def _mm(fa, fb, ka, kb, nm, nt, wo, wb):
    f = jnp.concatenate([fa[...], fb[...]], axis=1)
    k = jnp.concatenate([ka[...], kb[...]], axis=1)
    return (jnp.dot(nm[...] * f, wo[...], preferred_element_type=jnp.float32)
            + jnp.dot(nt[...] * k, wb[...], preferred_element_type=jnp.float32))


def _mm_stats_body(fa, fb, ka, kb, nm, nt, wo, wb, o_ref, st_ref):
    m = _mm(fa, fb, ka, kb, nm, nt, wo, wb)
    o_ref[...] = m
    _relu_stats(pl.program_id(0), jnp.maximum(m, 0.0), st_ref)


def _mm_res_body(fa, fb, ka, kb, nm, nt, wo, wb, x_ref, o_ref):
    o_ref[...] = x_ref[...] + _mm(fa, fb, ka, kb, nm, nt, wo, wb)


_half_spec = pl.BlockSpec((BLK, HALF), lambda i: (i, 0))
_norm_spec = pl.BlockSpec((BLK, 1), lambda i: (i, 0))
_w_spec = pl.BlockSpec((C, C), lambda i: (0, 0))
_full_spec = pl.BlockSpec((BLK, C), lambda i: (i, 0))

_mm_stats = pl.pallas_call(
    _mm_stats_body,
    grid=(NBLK,),
    in_specs=[_half_spec, _half_spec, _half_spec, _half_spec,
              _norm_spec, _norm_spec, _w_spec, _w_spec],
    out_specs=[_full_spec, pl.BlockSpec((8, C), lambda i: (0, 0))],
    out_shape=[jax.ShapeDtypeStruct((N_NODES, C), jnp.float32),
               jax.ShapeDtypeStruct((8, C), jnp.float32)],
)

_mm_res = pl.pallas_call(
    _mm_res_body,
    grid=(NBLK,),
    in_specs=[_half_spec, _half_spec, _half_spec, _half_spec,
              _norm_spec, _norm_spec, _w_spec, _w_spec, _full_spec],
    out_specs=_full_spec,
    out_shape=jax.ShapeDtypeStruct((N_NODES, C), jnp.float32),
)


def kernel(x, sources, targets, norm, norm_t, W1o, W1b, W2o, W2b):
    src2 = sources.reshape(N_EDGES // CHUNK, CHUNK)
    tgt2 = targets.reshape(N_EDGES // CHUNK, CHUNK)

    st0 = _stats(x)
    h_a, h_b = _bn_apply(x, st0)
    f_a, f_b, g_a, g_b = _sc_agg(h_a, h_b, src2, tgt2)
    m1, st1 = _mm_stats(f_a, f_b, g_a, g_b, norm, norm_t, W1o, W1b)
    h2_a, h2_b = _bn_apply(m1, st1)
    f2a, f2b, g2a, g2b = _sc_agg(h2_a, h2_b, src2, tgt2)
    return _mm_res(f2a, f2b, g2a, g2b, norm, norm_t, W2o, W2b, x)


# speedup vs baseline: 2.0529x; 2.0529x over previous
import jax
import jax.numpy as jnp
from jax import lax
from jax.experimental import pallas as pl
from jax.experimental.pallas import tpu as pltpu
from jax.experimental.pallas import tpu_sc as plsc

N_NODES = 50000
C = 64
W = 128
N_EDGES = 800000
EPS = 1e-5

NS = 16
CHUNK = 64
NBUF = 2
SCC = 32
E_PAD = 819200
CPT = E_PAD // NS // CHUNK
NSC = CPT // SCC
OWN = 12544
DUMMY = 64
ACC_ROWS = OWN + DUMMY
N_PAD = 4 * OWN
IPT = OWN // NS
PAD_SCT = 1 << 20


def _sc_agg_body(h, src_gat, tgt_sct, tgt_gat, src_sct, f_out, g_out,
                 gidx, sidx, rmp, rows, acc, sem_g):
    c = lax.axis_index("c")
    s = lax.axis_index("s")

    def phase(gat2, sct2, out_ref):
        def sweep(p, carry):
            base = pl.multiple_of((2 * p + c) * OWN, 8)
            ibase = pl.multiple_of(s * IPT, 8)
            pltpu.sync_copy(h.at[pl.ds(base + ibase, IPT)],
                            acc.at[pl.ds(ibase, IPT)])
            plsc.subcore_barrier()

            def superchunk(m, carry2):
                row0 = pl.multiple_of(s * CPT + m * SCC, 8)
                pltpu.sync_copy(gat2.at[pl.ds(row0, SCC)], gidx)
                pltpu.sync_copy(sct2.at[pl.ds(row0, SCC)], sidx)
                for b in range(NBUF):
                    pltpu.async_copy(h.at[gidx.at[b]], rows.at[b],
                                     sem_g.at[b])

                def chunk(jo, carry3):
                    for b in range(NBUF):
                        j = jo * NBUF + b
                        for k in range(CHUNK // 16):
                            t = sidx[j, pl.ds(k * 16, 16)]
                            local = t - base
                            ok = (local >= 0) & (local < OWN)
                            dummy = OWN + (t & (DUMMY - 1))
                            rmp[b, pl.ds(k * 16, 16)] = jnp.where(
                                ok, local, dummy)
                        pltpu.make_async_copy(
                            h.at[gidx.at[j]], rows.at[b], sem_g.at[b]).wait()
                        pltpu.sync_copy(rows.at[b], acc.at[rmp.at[b]],
                                        add=True)
                        jn = j + NBUF

                        @pl.when(jn < SCC)
                        def _():
                            pltpu.async_copy(h.at[gidx.at[jn]], rows.at[b],
                                             sem_g.at[b])
                    return carry3

                lax.fori_loop(0, SCC // NBUF, chunk, 0)
                return carry2

            lax.fori_loop(0, NSC, superchunk, 0)
            plsc.subcore_barrier()
            pltpu.sync_copy(acc.at[pl.ds(ibase, IPT)],
                            out_ref.at[pl.ds(base + ibase, IPT)])
            plsc.subcore_barrier()
            return carry

        lax.fori_loop(0, 2, sweep, 0)

    phase(src_gat, tgt_sct, f_out)
    phase(tgt_gat, src_sct, g_out)


_sc_agg = pl.kernel(
    _sc_agg_body,
    out_type=tuple(jax.ShapeDtypeStruct((N_PAD, W), jnp.float32)
                   for _ in range(2)),
    mesh=plsc.VectorSubcoreMesh(core_axis_name="c", subcore_axis_name="s"),
    scratch_types=[
        pltpu.VMEM((SCC, CHUNK), jnp.int32),
        pltpu.VMEM((SCC, CHUNK), jnp.int32),
        pltpu.VMEM((NBUF, CHUNK), jnp.int32),
        pltpu.VMEM((NBUF, CHUNK, W), jnp.float32),
        pltpu.VMEM_SHARED((ACC_ROWS, W), jnp.float32),
        pltpu.SemaphoreType.DMA((NBUF,)),
    ],
)


BLK = 2000
NBLK = N_NODES // BLK
BLKP = 1568
NBLKP = N_PAD // BLKP


def _relu_stats(i, r, st_ref):
    s1 = jnp.sum(r, axis=0, keepdims=True)
    s2 = jnp.sum(r * r, axis=0, keepdims=True)
    blk = jnp.concatenate(
        [s1, s2, jnp.zeros((6, C), jnp.float32)], axis=0)

    @pl.when(i == 0)
    def _():
        st_ref[...] = blk

    @pl.when(i != 0)
    def _():
        st_ref[...] = st_ref[...] + blk


def _stats_body(x_ref, st_ref):
    _relu_stats(pl.program_id(0), jnp.maximum(x_ref[...], 0.0), st_ref)


_stats = pl.pallas_call(
    _stats_body,
    grid=(NBLK,),
    in_specs=[pl.BlockSpec((BLK, C), lambda i: (i, 0))],
    out_specs=pl.BlockSpec((8, C), lambda i: (0, 0)),
    out_shape=jax.ShapeDtypeStruct((8, C), jnp.float32),
)


def _bn_body(x_ref, st_ref, h_ref):
    st = st_ref[...]
    mean = st[0:1, :] * (1.0 / N_NODES)
    var = st[1:2, :] * (1.0 / N_NODES) - mean * mean
    inv = lax.rsqrt(var + EPS)
    h = (jnp.maximum(x_ref[...], 0.0) - mean) * inv
    h_ref[...] = jnp.concatenate(
        [h, jnp.zeros((BLKP, W - C), jnp.float32)], axis=1)


_bn_apply = pl.pallas_call(
    _bn_body,
    grid=(NBLKP,),
    in_specs=[
        pl.BlockSpec((BLKP, C), lambda i: (i, 0)),
        pl.BlockSpec((8, C), lambda i: (0, 0)),
    ],
    out_specs=pl.BlockSpec((BLKP, W), lambda i: (i, 0)),
    out_shape=jax.ShapeDtypeStruct((N_PAD, W), jnp.float32),
)


def _mm(f_ref, g_ref, nm, nt, wo, wb):
    f = f_ref[...][:, :C]
    g = g_ref[...][:, :C]
    return (jnp.dot(nm[...] * f, wo[...], preferred_element_type=jnp.float32)
            + jnp.dot(nt[...] * g, wb[...], preferred_element_type=jnp.float32))


def _mm_stats_body(f_ref, g_ref, nm, nt, wo, wb, o_ref, st_ref):
    m = _mm(f_ref, g_ref, nm, nt, wo, wb)
    o_ref[...] = m
    _relu_stats(pl.program_id(0), jnp.maximum(m, 0.0), st_ref)


def _mm_res_body(f_ref, g_ref, nm, nt, wo, wb, x_ref, o_ref):
    o_ref[...] = x_ref[...] + _mm(f_ref, g_ref, nm, nt, wo, wb)


_fg_spec = pl.BlockSpec((BLK, W), lambda i: (i, 0))
_norm_spec = pl.BlockSpec((BLK, 1), lambda i: (i, 0))
_w_spec = pl.BlockSpec((C, C), lambda i: (0, 0))
_full_spec = pl.BlockSpec((BLK, C), lambda i: (i, 0))

_mm_stats = pl.pallas_call(
    _mm_stats_body,
    grid=(NBLK,),
    in_specs=[_fg_spec, _fg_spec, _norm_spec, _norm_spec, _w_spec, _w_spec],
    out_specs=[_full_spec, pl.BlockSpec((8, C), lambda i: (0, 0))],
    out_shape=[jax.ShapeDtypeStruct((N_NODES, C), jnp.float32),
               jax.ShapeDtypeStruct((8, C), jnp.float32)],
)

_mm_res = pl.pallas_call(
    _mm_res_body,
    grid=(NBLK,),
    in_specs=[_fg_spec, _fg_spec, _norm_spec, _norm_spec, _w_spec, _w_spec,
              _full_spec],
    out_specs=_full_spec,
    out_shape=jax.ShapeDtypeStruct((N_NODES, C), jnp.float32),
)


def kernel(x, sources, targets, norm, norm_t, W1o, W1b, W2o, W2b):
    npad = E_PAD - N_EDGES
    seq = jnp.arange(npad, dtype=jnp.int32)
    gpad = seq % N_NODES
    spad = jnp.full((npad,), PAD_SCT, jnp.int32)
    shape2 = (E_PAD // CHUNK, CHUNK)
    src_gat = jnp.concatenate([sources, gpad]).reshape(shape2)
    tgt_sct = jnp.concatenate([targets, spad]).reshape(shape2)
    tgt_gat = jnp.concatenate([targets, gpad]).reshape(shape2)
    src_sct = jnp.concatenate([sources, spad]).reshape(shape2)

    st0 = _stats(x)
    h1 = _bn_apply(x, st0)
    f1, g1 = _sc_agg(h1, src_gat, tgt_sct, tgt_gat, src_sct)
    m1, st1 = _mm_stats(f1, g1, norm, norm_t, W1o, W1b)
    h2 = _bn_apply(m1, st1)
    f2, g2 = _sc_agg(h2, src_gat, tgt_sct, tgt_gat, src_sct)
    return _mm_res(f2, g2, norm, norm_t, W2o, W2b, x)


# speedup vs baseline: 5.9651x; 2.9057x over previous
import jax
import jax.numpy as jnp
from jax import lax
from jax.experimental import pallas as pl
from jax.experimental.pallas import tpu as pltpu
from jax.experimental.pallas import tpu_sc as plsc

N_NODES = 50000
C = 64
W = 128
N_EDGES = 800000
EPS = 1e-5

NS = 16
CHUNK = 64
NBUF = 2
SCC = 32
E_PAD = 819200
CPT = E_PAD // NS // CHUNK
NSC = CPT // SCC
OWN = 12544
DUMMY = 64
ACC_ROWS = OWN + DUMMY
N_PAD = 4 * OWN
IPT = OWN // NS
PAD_SCT = 1 << 20


def _sc_agg_body(h, src_gat, tgt_sct, tgt_gat, src_sct, f_out, g_out,
                 gidx, sidx, fifo_g, fifo_s, rows, acc, sem_g, sem_s):
    c = lax.axis_index("c")
    s = lax.axis_index("s")
    lanes = lax.iota(jnp.int32, 16)

    def fire(n, rowbuf, sembuf_g, sembuf_s, orowbuf, osembuf_g, osembuf_s):
        @pl.when(n >= 2)
        def _():
            pltpu.make_async_copy(rowbuf, acc.at[fifo_s.at[(n - 2) & 3]],
                                  sembuf_s).wait()
        pltpu.async_copy(h.at[fifo_g.at[n & 3]], rowbuf, sembuf_g)

        @pl.when(n >= 1)
        def _():
            pltpu.make_async_copy(h.at[fifo_g.at[(n - 1) & 3]], orowbuf,
                                  osembuf_g).wait()
            pltpu.async_copy(orowbuf, acc.at[fifo_s.at[(n - 1) & 3]],
                             osembuf_s, add=True)

    def fire_either(n):
        @pl.when((n & 1) == 0)
        def _():
            fire(n, rows.at[0], sem_g.at[0], sem_s.at[0],
                 rows.at[1], sem_g.at[1], sem_s.at[1])

        @pl.when((n & 1) == 1)
        def _():
            fire(n, rows.at[1], sem_g.at[1], sem_s.at[1],
                 rows.at[0], sem_g.at[0], sem_s.at[0])

    def phase(gat2, sct2, out_ref):
        def sweep(p, carry0):
            base = pl.multiple_of((2 * p + c) * OWN, 8)
            ibase = pl.multiple_of(s * IPT, 8)
            pltpu.sync_copy(h.at[pl.ds(base + ibase, IPT)],
                            acc.at[pl.ds(ibase, IPT)])
            plsc.subcore_barrier()

            def superchunk(m, carry):
                row0 = pl.multiple_of(s * CPT + m * SCC, 8)
                pltpu.sync_copy(gat2.at[pl.ds(row0, SCC)], gidx)
                pltpu.sync_copy(sct2.at[pl.ds(row0, SCC)], sidx)

                def chunk(j, carry2):
                    fill, nfired = carry2
                    for k in range(CHUNK // 16):
                        t = sidx[j, pl.ds(k * 16, 16)]
                        g = gidx[j, pl.ds(k * 16, 16)]
                        local = t - base
                        ok = (local >= 0) & (local < OWN)
                        pos = fill + plsc.cumsum(ok.astype(jnp.int32)) - 1
                        plsc.store_scatter(
                            fifo_g, [(pos >> 6) & 3, pos & 63], g, mask=ok)
                        plsc.store_scatter(
                            fifo_s, [(pos >> 6) & 3, pos & 63], local,
                            mask=ok)
                        fill = fill + jnp.sum(ok.astype(jnp.int32))
                    do_fire = fill >= (nfired + 1) * 64

                    @pl.when(do_fire)
                    def _():
                        fire_either(nfired)

                    nfired = jnp.where(do_fire, nfired + 1, nfired)
                    return fill, nfired

                return lax.fori_loop(0, SCC, chunk, carry)

            fill, nfired = lax.fori_loop(0, NSC, superchunk,
                                         (jnp.int32(0), jnp.int32(0)))

            rem = fill - nfired * 64
            for k in range(4):
                posk = k * 16 + lanes
                padmask = posk >= rem
                plsc.store_scatter(fifo_g, [jnp.full((16,), nfired & 3,
                                                     jnp.int32), posk],
                                   posk + s, mask=padmask)
                plsc.store_scatter(fifo_s, [jnp.full((16,), nfired & 3,
                                                     jnp.int32), posk],
                                   OWN + posk, mask=padmask)
            fire_either(nfired)

            @pl.when((nfired & 1) == 0)
            def _():
                pltpu.make_async_copy(h.at[fifo_g.at[nfired & 3]],
                                      rows.at[0], sem_g.at[0]).wait()
                pltpu.sync_copy(rows.at[0], acc.at[fifo_s.at[nfired & 3]],
                                add=True)

                @pl.when(nfired >= 1)
                def _():
                    pltpu.make_async_copy(
                        rows.at[1], acc.at[fifo_s.at[(nfired - 1) & 3]],
                        sem_s.at[1]).wait()

            @pl.when((nfired & 1) == 1)
            def _():
                pltpu.make_async_copy(h.at[fifo_g.at[nfired & 3]],
                                      rows.at[1], sem_g.at[1]).wait()
                pltpu.sync_copy(rows.at[1], acc.at[fifo_s.at[nfired & 3]],
                                add=True)
                pltpu.make_async_copy(
                    rows.at[0], acc.at[fifo_s.at[(nfired - 1) & 3]],
                    sem_s.at[0]).wait()

            plsc.subcore_barrier()
            pltpu.sync_copy(acc.at[pl.ds(ibase, IPT)],
                            out_ref.at[pl.ds(base + ibase, IPT)])
            plsc.subcore_barrier()
            return carry0

        lax.fori_loop(0, 2, sweep, 0)

    phase(src_gat, tgt_sct, f_out)
    phase(tgt_gat, src_sct, g_out)


_sc_agg = pl.kernel(
    _sc_agg_body,
    out_type=tuple(jax.ShapeDtypeStruct((N_PAD, W), jnp.float32)
                   for _ in range(2)),
    mesh=plsc.VectorSubcoreMesh(core_axis_name="c", subcore_axis_name="s"),
    compiler_params=pltpu.CompilerParams(needs_layout_passes=False),
    scratch_types=[
        pltpu.VMEM((SCC, CHUNK), jnp.int32),
        pltpu.VMEM((SCC, CHUNK), jnp.int32),
        pltpu.VMEM((4, 64), jnp.int32),
        pltpu.VMEM((4, 64), jnp.int32),
        pltpu.VMEM((2, 64, W), jnp.float32),
        pltpu.VMEM_SHARED((ACC_ROWS, W), jnp.float32),
        pltpu.SemaphoreType.DMA((2,)),
        pltpu.SemaphoreType.DMA((2,)),
    ],
)


BLK = 2000
NBLK = N_NODES // BLK
BLKP = 1568
NBLKP = N_PAD // BLKP


def _relu_stats(i, r, st_ref):
    s1 = jnp.sum(r, axis=0, keepdims=True)
    s2 = jnp.sum(r * r, axis=0, keepdims=True)
    blk = jnp.concatenate(
        [s1, s2, jnp.zeros((6, C), jnp.float32)], axis=0)

    @pl.when(i == 0)
    def _():
        st_ref[...] = blk

    @pl.when(i != 0)
    def _():
        st_ref[...] = st_ref[...] + blk


def _stats_body(x_ref, st_ref):
    _relu_stats(pl.program_id(0), jnp.maximum(x_ref[...], 0.0), st_ref)


_stats = pl.pallas_call(
    _stats_body,
    grid=(NBLK,),
    in_specs=[pl.BlockSpec((BLK, C), lambda i: (i, 0))],
    out_specs=pl.BlockSpec((8, C), lambda i: (0, 0)),
    out_shape=jax.ShapeDtypeStruct((8, C), jnp.float32),
)


def _bn_body(x_ref, st_ref, h_ref):
    st = st_ref[...]
    mean = st[0:1, :] * (1.0 / N_NODES)
    var = st[1:2, :] * (1.0 / N_NODES) - mean * mean
    inv = lax.rsqrt(var + EPS)
    h = (jnp.maximum(x_ref[...], 0.0) - mean) * inv
    h_ref[...] = jnp.concatenate(
        [h, jnp.zeros((BLKP, W - C), jnp.float32)], axis=1)


_bn_apply = pl.pallas_call(
    _bn_body,
    grid=(NBLKP,),
    in_specs=[
        pl.BlockSpec((BLKP, C), lambda i: (i, 0)),
        pl.BlockSpec((8, C), lambda i: (0, 0)),
    ],
    out_specs=pl.BlockSpec((BLKP, W), lambda i: (i, 0)),
    out_shape=jax.ShapeDtypeStruct((N_PAD, W), jnp.float32),
)


def _mm(f_ref, g_ref, nm, nt, wo, wb):
    f = f_ref[...][:, :C]
    g = g_ref[...][:, :C]
    return (jnp.dot(nm[...] * f, wo[...], preferred_element_type=jnp.float32)
            + jnp.dot(nt[...] * g, wb[...], preferred_element_type=jnp.float32))


def _mm_stats_body(f_ref, g_ref, nm, nt, wo, wb, o_ref, st_ref):
    m = _mm(f_ref, g_ref, nm, nt, wo, wb)
    o_ref[...] = m
    _relu_stats(pl.program_id(0), jnp.maximum(m, 0.0), st_ref)


def _mm_res_body(f_ref, g_ref, nm, nt, wo, wb, x_ref, o_ref):
    o_ref[...] = x_ref[...] + _mm(f_ref, g_ref, nm, nt, wo, wb)


_fg_spec = pl.BlockSpec((BLK, W), lambda i: (i, 0))
_norm_spec = pl.BlockSpec((BLK, 1), lambda i: (i, 0))
_w_spec = pl.BlockSpec((C, C), lambda i: (0, 0))
_full_spec = pl.BlockSpec((BLK, C), lambda i: (i, 0))

_mm_stats = pl.pallas_call(
    _mm_stats_body,
    grid=(NBLK,),
    in_specs=[_fg_spec, _fg_spec, _norm_spec, _norm_spec, _w_spec, _w_spec],
    out_specs=[_full_spec, pl.BlockSpec((8, C), lambda i: (0, 0))],
    out_shape=[jax.ShapeDtypeStruct((N_NODES, C), jnp.float32),
               jax.ShapeDtypeStruct((8, C), jnp.float32)],
)

_mm_res = pl.pallas_call(
    _mm_res_body,
    grid=(NBLK,),
    in_specs=[_fg_spec, _fg_spec, _norm_spec, _norm_spec, _w_spec, _w_spec,
              _full_spec],
    out_specs=_full_spec,
    out_shape=jax.ShapeDtypeStruct((N_NODES, C), jnp.float32),
)


def kernel(x, sources, targets, norm, norm_t, W1o, W1b, W2o, W2b):
    npad = E_PAD - N_EDGES
    seq = jnp.arange(npad, dtype=jnp.int32)
    gpad = seq % N_NODES
    spad = jnp.full((npad,), PAD_SCT, jnp.int32)
    shape2 = (E_PAD // CHUNK, CHUNK)
    src_gat = jnp.concatenate([sources, gpad]).reshape(shape2)
    tgt_sct = jnp.concatenate([targets, spad]).reshape(shape2)
    tgt_gat = jnp.concatenate([targets, gpad]).reshape(shape2)
    src_sct = jnp.concatenate([sources, spad]).reshape(shape2)

    st0 = _stats(x)
    h1 = _bn_apply(x, st0)
    f1, g1 = _sc_agg(h1, src_gat, tgt_sct, tgt_gat, src_sct)
    m1, st1 = _mm_stats(f1, g1, norm, norm_t, W1o, W1b)
    h2 = _bn_apply(m1, st1)
    f2, g2 = _sc_agg(h2, src_gat, tgt_sct, tgt_gat, src_sct)
    return _mm_res(f2, g2, norm, norm_t, W2o, W2b, x)


# speedup vs baseline: 6.6653x; 1.1174x over previous
import jax
import jax.numpy as jnp
from jax import lax
from jax.experimental import pallas as pl
from jax.experimental.pallas import tpu as pltpu
from jax.experimental.pallas import tpu_sc as plsc

N_NODES = 50000
C = 64
W = 128
N_EDGES = 800000
EPS = 1e-5

NS = 16
CHUNK = 64
NBUF = 2
SCC = 80
E_PAD = 819200
CPT = E_PAD // NS // CHUNK
NSC = CPT // SCC
OWN = 12544
DUMMY = 64
ACC_ROWS = OWN + DUMMY
N_PAD = 4 * OWN
IPT = OWN // NS
PAD_SCT = 1 << 20


def _sc_agg_body(h, comb_f, comb_b, f_out, g_out,
                 cidx, fifo_g, fifo_s, rows, acc, sem_g, sem_s):
    c = lax.axis_index("c")
    s = lax.axis_index("s")
    lanes = lax.iota(jnp.int32, 16)

    def fire(n, rowbuf, sembuf_g, sembuf_s, orowbuf, osembuf_g, osembuf_s):
        @pl.when(n >= 2)
        def _():
            pltpu.make_async_copy(rowbuf, acc.at[fifo_s.at[(n - 2) & 3]],
                                  sembuf_s).wait()
        pltpu.async_copy(h.at[fifo_g.at[n & 3]], rowbuf, sembuf_g)

        @pl.when(n >= 1)
        def _():
            pltpu.make_async_copy(h.at[fifo_g.at[(n - 1) & 3]], orowbuf,
                                  osembuf_g).wait()
            pltpu.async_copy(orowbuf, acc.at[fifo_s.at[(n - 1) & 3]],
                             osembuf_s, add=True)

    def fire_either(n):
        @pl.when((n & 1) == 0)
        def _():
            fire(n, rows.at[0], sem_g.at[0], sem_s.at[0],
                 rows.at[1], sem_g.at[1], sem_s.at[1])

        @pl.when((n & 1) == 1)
        def _():
            fire(n, rows.at[1], sem_g.at[1], sem_s.at[1],
                 rows.at[0], sem_g.at[0], sem_s.at[0])

    def phase(comb, out_ref):
        def sweep(p, carry0):
            base = pl.multiple_of((2 * p + c) * OWN, 8)
            ibase = pl.multiple_of(s * IPT, 8)
            pltpu.sync_copy(h.at[pl.ds(base + ibase, IPT)],
                            acc.at[pl.ds(ibase, IPT)])
            plsc.subcore_barrier()

            def superchunk(m, carry):
                row0 = pl.multiple_of(s * CPT + m * SCC, 8)
                pltpu.sync_copy(comb.at[pl.ds(row0, SCC)], cidx)

                def chunk(j, carry2):
                    fill, nfired = carry2
                    for k in range(CHUNK // 16):
                        t = cidx[j, pl.ds(CHUNK + k * 16, 16)]
                        g = cidx[j, pl.ds(k * 16, 16)]
                        local = t - base
                        ok = (local >= 0) & (local < OWN)
                        pos = fill + plsc.cumsum(ok.astype(jnp.int32)) - 1
                        plsc.store_scatter(
                            fifo_g, [(pos >> 6) & 3, pos & 63], g, mask=ok)
                        plsc.store_scatter(
                            fifo_s, [(pos >> 6) & 3, pos & 63], local,
                            mask=ok)
                        fill = fill + jnp.sum(ok.astype(jnp.int32))
                    do_fire = fill >= (nfired + 1) * 64

                    @pl.when(do_fire)
                    def _():
                        fire_either(nfired)

                    nfired = jnp.where(do_fire, nfired + 1, nfired)
                    return fill, nfired

                return lax.fori_loop(0, SCC, chunk, carry)

            fill, nfired = lax.fori_loop(0, NSC, superchunk,
                                         (jnp.int32(0), jnp.int32(0)))

            rem = fill - nfired * 64
            for k in range(4):
                posk = k * 16 + lanes
                padmask = posk >= rem
                plsc.store_scatter(fifo_g, [jnp.full((16,), nfired & 3,
                                                     jnp.int32), posk],
                                   posk + s, mask=padmask)
                plsc.store_scatter(fifo_s, [jnp.full((16,), nfired & 3,
                                                     jnp.int32), posk],
                                   OWN + posk, mask=padmask)
            fire_either(nfired)

            @pl.when((nfired & 1) == 0)
            def _():
                pltpu.make_async_copy(h.at[fifo_g.at[nfired & 3]],
                                      rows.at[0], sem_g.at[0]).wait()
                pltpu.sync_copy(rows.at[0], acc.at[fifo_s.at[nfired & 3]],
                                add=True)

                @pl.when(nfired >= 1)
                def _():
                    pltpu.make_async_copy(
                        rows.at[1], acc.at[fifo_s.at[(nfired - 1) & 3]],
                        sem_s.at[1]).wait()

            @pl.when((nfired & 1) == 1)
            def _():
                pltpu.make_async_copy(h.at[fifo_g.at[nfired & 3]],
                                      rows.at[1], sem_g.at[1]).wait()
                pltpu.sync_copy(rows.at[1], acc.at[fifo_s.at[nfired & 3]],
                                add=True)
                pltpu.make_async_copy(
                    rows.at[0], acc.at[fifo_s.at[(nfired - 1) & 3]],
                    sem_s.at[0]).wait()

            plsc.subcore_barrier()
            pltpu.sync_copy(acc.at[pl.ds(ibase, IPT)],
                            out_ref.at[pl.ds(base + ibase, IPT)])
            plsc.subcore_barrier()
            return carry0

        lax.fori_loop(0, 2, sweep, 0)

    phase(comb_f, f_out)
    phase(comb_b, g_out)


_sc_agg = pl.kernel(
    _sc_agg_body,
    out_type=tuple(jax.ShapeDtypeStruct((N_PAD, W), jnp.float32)
                   for _ in range(2)),
    mesh=plsc.VectorSubcoreMesh(core_axis_name="c", subcore_axis_name="s"),
    compiler_params=pltpu.CompilerParams(needs_layout_passes=False),
    scratch_types=[
        pltpu.VMEM((SCC, 2 * CHUNK), jnp.int32),
        pltpu.VMEM((4, 64), jnp.int32),
        pltpu.VMEM((4, 64), jnp.int32),
        pltpu.VMEM((2, 64, W), jnp.float32),
        pltpu.VMEM_SHARED((ACC_ROWS, W), jnp.float32),
        pltpu.SemaphoreType.DMA((2,)),
        pltpu.SemaphoreType.DMA((2,)),
    ],
)


BLK = 2000
NBLK = N_NODES // BLK
BLKP = 1568
NBLKP = N_PAD // BLKP


def _relu_stats(i, r, st_ref):
    s1 = jnp.sum(r, axis=0, keepdims=True)
    s2 = jnp.sum(r * r, axis=0, keepdims=True)
    blk = jnp.concatenate(
        [s1, s2, jnp.zeros((6, C), jnp.float32)], axis=0)

    @pl.when(i == 0)
    def _():
        st_ref[...] = blk

    @pl.when(i != 0)
    def _():
        st_ref[...] = st_ref[...] + blk


def _stats_body(x_ref, st_ref):
    _relu_stats(pl.program_id(0), jnp.maximum(x_ref[...], 0.0), st_ref)


_stats = pl.pallas_call(
    _stats_body,
    grid=(NBLK,),
    in_specs=[pl.BlockSpec((BLK, C), lambda i: (i, 0))],
    out_specs=pl.BlockSpec((8, C), lambda i: (0, 0)),
    out_shape=jax.ShapeDtypeStruct((8, C), jnp.float32),
)


def _bn_body(x_ref, st_ref, h_ref):
    st = st_ref[...]
    mean = st[0:1, :] * (1.0 / N_NODES)
    var = st[1:2, :] * (1.0 / N_NODES) - mean * mean
    inv = lax.rsqrt(var + EPS)
    h = (jnp.maximum(x_ref[...], 0.0) - mean) * inv
    h_ref[...] = jnp.concatenate(
        [h, jnp.zeros((BLKP, W - C), jnp.float32)], axis=1)


_bn_apply = pl.pallas_call(
    _bn_body,
    grid=(NBLKP,),
    in_specs=[
        pl.BlockSpec((BLKP, C), lambda i: (i, 0)),
        pl.BlockSpec((8, C), lambda i: (0, 0)),
    ],
    out_specs=pl.BlockSpec((BLKP, W), lambda i: (i, 0)),
    out_shape=jax.ShapeDtypeStruct((N_PAD, W), jnp.float32),
)


def _mm(f_ref, g_ref, nm, nt, wo, wb):
    f = f_ref[...][:, :C]
    g = g_ref[...][:, :C]
    return (jnp.dot(nm[...] * f, wo[...], preferred_element_type=jnp.float32)
            + jnp.dot(nt[...] * g, wb[...], preferred_element_type=jnp.float32))


def _mm_stats_body(f_ref, g_ref, nm, nt, wo, wb, o_ref, st_ref):
    m = _mm(f_ref, g_ref, nm, nt, wo, wb)
    o_ref[...] = m
    _relu_stats(pl.program_id(0), jnp.maximum(m, 0.0), st_ref)


def _mm_res_body(f_ref, g_ref, nm, nt, wo, wb, x_ref, o_ref):
    o_ref[...] = x_ref[...] + _mm(f_ref, g_ref, nm, nt, wo, wb)


_fg_spec = pl.BlockSpec((BLK, W), lambda i: (i, 0))
_norm_spec = pl.BlockSpec((BLK, 1), lambda i: (i, 0))
_w_spec = pl.BlockSpec((C, C), lambda i: (0, 0))
_full_spec = pl.BlockSpec((BLK, C), lambda i: (i, 0))

_mm_stats = pl.pallas_call(
    _mm_stats_body,
    grid=(NBLK,),
    in_specs=[_fg_spec, _fg_spec, _norm_spec, _norm_spec, _w_spec, _w_spec],
    out_specs=[_full_spec, pl.BlockSpec((8, C), lambda i: (0, 0))],
    out_shape=[jax.ShapeDtypeStruct((N_NODES, C), jnp.float32),
               jax.ShapeDtypeStruct((8, C), jnp.float32)],
)

_mm_res = pl.pallas_call(
    _mm_res_body,
    grid=(NBLK,),
    in_specs=[_fg_spec, _fg_spec, _norm_spec, _norm_spec, _w_spec, _w_spec,
              _full_spec],
    out_specs=_full_spec,
    out_shape=jax.ShapeDtypeStruct((N_NODES, C), jnp.float32),
)


def kernel(x, sources, targets, norm, norm_t, W1o, W1b, W2o, W2b):
    npad = E_PAD - N_EDGES
    seq = jnp.arange(npad, dtype=jnp.int32)
    gpad = seq % N_NODES
    spad = jnp.full((npad,), PAD_SCT, jnp.int32)
    shape2 = (E_PAD // CHUNK, CHUNK)
    src_gat = jnp.concatenate([sources, gpad]).reshape(shape2)
    tgt_sct = jnp.concatenate([targets, spad]).reshape(shape2)
    tgt_gat = jnp.concatenate([targets, gpad]).reshape(shape2)
    src_sct = jnp.concatenate([sources, spad]).reshape(shape2)
    comb_f = jnp.concatenate([src_gat, tgt_sct], axis=1)
    comb_b = jnp.concatenate([tgt_gat, src_sct], axis=1)

    st0 = _stats(x)
    h1 = _bn_apply(x, st0)
    f1, g1 = _sc_agg(h1, comb_f, comb_b)
    m1, st1 = _mm_stats(f1, g1, norm, norm_t, W1o, W1b)
    h2 = _bn_apply(m1, st1)
    f2, g2 = _sc_agg(h2, comb_f, comb_b)
    return _mm_res(f2, g2, norm, norm_t, W2o, W2b, x)
